# baseline (speedup 1.0000x reference)
import jax
import jax.numpy as jnp
from jax.experimental import pallas as pl
from jax.experimental.pallas import tpu as pltpu

_C = 384
_RED = 16
_D = max(_C // _RED, 4)
_B = 16
_H = 28
_W = 28
_HW = _H * _W
_TOPK = int(_C * 0.15)
_HALF = 2
_CHUNK = 98
_NCHUNK = _HW // _CHUNK


def _targetdrop_kernel(xt_ref, w1_ref, w2_ref, out_ref,
                       xs_ref, ssum_ref, smax_ref, sidx_ref,
                       scale0_ref, mhm2_ref, mwm2_ref):
    j = pl.program_id(0)

    @pl.when(j < _NCHUNK)
    def _stats():
        xc = xt_ref[...]
        xs_ref[pl.ds(j * _CHUNK, _CHUNK)] = xc

        csum = jnp.sum(xc, axis=0)
        cmax = jnp.max(xc, axis=0)
        pidx = (jax.lax.broadcasted_iota(jnp.int32, (_CHUNK, 1, 1), 0)
                + j * _CHUNK)
        cidx = jnp.min(jnp.where(xc == cmax[None], pidx, _HW),
                       axis=0)

        @pl.when(j == 0)
        def _():
            ssum_ref[...] = csum
            smax_ref[...] = cmax
            sidx_ref[...] = cidx

        @pl.when(j > 0)
        def _():
            upd = cmax > smax_ref[...]
            sidx_ref[...] = jnp.where(upd, cidx, sidx_ref[...])
            smax_ref[...] = jnp.maximum(smax_ref[...], cmax)
            ssum_ref[...] = ssum_ref[...] + csum

        @pl.when(j == _NCHUNK - 1)
        def _():
            pooled = ssum_ref[...] * (1.0 / _HW)
            hid = jax.lax.dot_general(
                pooled, w1_ref[...], (((1,), (1,)), ((), ())),
                preferred_element_type=jnp.float32)
            hid = jnp.maximum(hid, 0.0)
            m = jax.nn.sigmoid(jax.lax.dot_general(
                hid, w2_ref[...], (((1,), (1,)), ((), ())),
                preferred_element_type=jnp.float32))

            bits = jax.lax.bitcast_convert_type(m, jnp.int32)
            p = jnp.zeros((_B, 1), jnp.int32)
            for k in range(29, -1, -1):
                t = p | (1 << k)
                cnt = jnp.sum((bits >= t).astype(jnp.int32), axis=1,
                              keepdims=True)
                p = jnp.where(cnt >= _TOPK, t, p)
            gt = bits > p
            eq = bits == p
            n_gt = jnp.sum(gt.astype(jnp.int32), axis=1, keepdims=True)
            needed = _TOPK - n_gt
            idx = jax.lax.broadcasted_iota(jnp.int32, (_B, _C), 1)
            s = jnp.zeros((_B, 1), jnp.int32)
            for k in range(8, -1, -1):
                t2 = s | (1 << k)
                cnt2 = jnp.sum((eq & (idx >= t2)).astype(jnp.int32),
                               axis=1, keepdims=True)
                s = jnp.where(cnt2 >= needed, t2, s)
            selected = gt | (eq & (idx >= s))

            amax = sidx_ref[...]
            mh = amax // _W
            mw = amax - mh * _W
            h1 = jnp.maximum(mh - _HALF, 0)
            h2 = jnp.minimum(mh + _HALF, _H - 1)
            w1 = jnp.maximum(mw - _HALF, 0)
            w2 = jnp.minimum(mw + _HALF, _W - 1)
            nzero = (h2 - h1 + 1) * (w2 - w1 + 1)
            lam = _HW / (_HW - nzero.astype(jnp.float32))
            scale0_ref[...] = jnp.where(selected, lam, 1.0)
            mhm2_ref[...] = jnp.where(selected, mh - _HALF, 10 * _H)
            mwm2_ref[...] = mw - _HALF

    @pl.when(j >= _NCHUNK)
    def _apply():
        jp = j - _NCHUNK
        xc = xs_ref[pl.ds(jp * _CHUNK, _CHUNK)]
        pidx = (jax.lax.broadcasted_iota(jnp.int32, (_CHUNK, 1, 1), 0)
                + jp * _CHUNK)
        rj = pidx // _W
        cj = pidx - rj * _W
        in_h = (rj - mhm2_ref[...][None]).astype(jnp.uint32) <= 2 * _HALF
        in_w = (cj - mwm2_ref[...][None]).astype(jnp.uint32) <= 2 * _HALF
        drop = in_h & in_w
        out_ref[...] = jnp.where(drop, 0.0, xc * scale0_ref[...][None])


def kernel(x, W1, W2):
    B, C, H, W = x.shape
    xt = jnp.transpose(x.reshape(B, C, H * W), (2, 0, 1))

    out_t = pl.pallas_call(
        _targetdrop_kernel,
        grid=(2 * _NCHUNK,),
        in_specs=[
            pl.BlockSpec((_CHUNK, B, C),
                         lambda j: (jnp.minimum(j, _NCHUNK - 1), 0, 0)),
            pl.BlockSpec((_D, C), lambda j: (0, 0)),
            pl.BlockSpec((C, _D), lambda j: (0, 0)),
        ],
        out_specs=pl.BlockSpec((_CHUNK, B, C),
                               lambda j: (jnp.maximum(j - _NCHUNK, 0),
                                          0, 0)),
        out_shape=jax.ShapeDtypeStruct((H * W, B, C), jnp.float32),
        scratch_shapes=[
            pltpu.VMEM((_HW, B, C), jnp.float32),
            pltpu.VMEM((B, C), jnp.float32),
            pltpu.VMEM((B, C), jnp.float32),
            pltpu.VMEM((B, C), jnp.int32),
            pltpu.VMEM((B, C), jnp.float32),
            pltpu.VMEM((B, C), jnp.int32),
            pltpu.VMEM((B, C), jnp.int32),
        ],
        compiler_params=pltpu.CompilerParams(
            dimension_semantics=("arbitrary",)),
    )(xt, W1, W2)

    return jnp.transpose(out_t, (1, 2, 0)).reshape(B, C, H, W)


# speedup vs baseline: 1.0889x; 1.0889x over previous
import jax
import jax.numpy as jnp
from jax.experimental import pallas as pl
from jax.experimental.pallas import tpu as pltpu

_C = 384
_RED = 16
_D = max(_C // _RED, 4)
_B = 16
_H = 28
_W = 28
_HW = _H * _W
_TOPK = int(_C * 0.15)
_HALF = 2
_CHUNK = 196
_NCHUNK = _HW // _CHUNK


def _targetdrop_kernel(xt_ref, w1_ref, w2_ref, out_ref,
                       xs_ref, ssum_ref, smax_ref, sidx_ref,
                       scale0_ref, mhm2_ref, mwm2_ref):
    j = pl.program_id(0)

    @pl.when(j < _NCHUNK)
    def _stats():
        xc = xt_ref[...]
        xs_ref[pl.ds(j * _CHUNK, _CHUNK)] = xc

        csum = jnp.sum(xc, axis=0)
        cmax = jnp.max(xc, axis=0)
        pidx = (jax.lax.broadcasted_iota(jnp.int32, (_CHUNK, 1, 1), 0)
                + j * _CHUNK)
        cidx = jnp.min(jnp.where(xc == cmax[None], pidx, _HW),
                       axis=0)

        @pl.when(j == 0)
        def _():
            ssum_ref[...] = csum
            smax_ref[...] = cmax
            sidx_ref[...] = cidx

        @pl.when(j > 0)
        def _():
            upd = cmax > smax_ref[...]
            sidx_ref[...] = jnp.where(upd, cidx, sidx_ref[...])
            smax_ref[...] = jnp.maximum(smax_ref[...], cmax)
            ssum_ref[...] = ssum_ref[...] + csum

        @pl.when(j == _NCHUNK - 1)
        def _():
            pooled = ssum_ref[...] * (1.0 / _HW)
            hid = jax.lax.dot_general(
                pooled, w1_ref[...], (((1,), (1,)), ((), ())),
                preferred_element_type=jnp.float32)
            hid = jnp.maximum(hid, 0.0)
            m = jax.nn.sigmoid(jax.lax.dot_general(
                hid, w2_ref[...], (((1,), (1,)), ((), ())),
                preferred_element_type=jnp.float32))

            bits = jax.lax.bitcast_convert_type(m, jnp.int32)
            p = jnp.zeros((_B, 1), jnp.int32)
            for k in range(29, -1, -1):
                t = p | (1 << k)
                cnt = jnp.sum((bits >= t).astype(jnp.int32), axis=1,
                              keepdims=True)
                p = jnp.where(cnt >= _TOPK, t, p)
            gt = bits > p
            eq = bits == p
            n_gt = jnp.sum(gt.astype(jnp.int32), axis=1, keepdims=True)
            needed = _TOPK - n_gt
            idx = jax.lax.broadcasted_iota(jnp.int32, (_B, _C), 1)
            s = jnp.zeros((_B, 1), jnp.int32)
            for k in range(8, -1, -1):
                t2 = s | (1 << k)
                cnt2 = jnp.sum((eq & (idx >= t2)).astype(jnp.int32),
                               axis=1, keepdims=True)
                s = jnp.where(cnt2 >= needed, t2, s)
            selected = gt | (eq & (idx >= s))

            amax = sidx_ref[...]
            mh = amax // _W
            mw = amax - mh * _W
            h1 = jnp.maximum(mh - _HALF, 0)
            h2 = jnp.minimum(mh + _HALF, _H - 1)
            w1 = jnp.maximum(mw - _HALF, 0)
            w2 = jnp.minimum(mw + _HALF, _W - 1)
            nzero = (h2 - h1 + 1) * (w2 - w1 + 1)
            lam = _HW / (_HW - nzero.astype(jnp.float32))
            scale0_ref[...] = jnp.where(selected, lam, 1.0)
            mhm2_ref[...] = jnp.where(selected, mh - _HALF, 10 * _H)
            mwm2_ref[...] = mw - _HALF

    @pl.when(j >= _NCHUNK)
    def _apply():
        jp = j - _NCHUNK
        xc = xs_ref[pl.ds(jp * _CHUNK, _CHUNK)]
        pidx = (jax.lax.broadcasted_iota(jnp.int32, (_CHUNK, 1, 1), 0)
                + jp * _CHUNK)
        rj = pidx // _W
        cj = pidx - rj * _W
        in_h = (rj - mhm2_ref[...][None]).astype(jnp.uint32) <= 2 * _HALF
        in_w = (cj - mwm2_ref[...][None]).astype(jnp.uint32) <= 2 * _HALF
        drop = in_h & in_w
        out_ref[...] = jnp.where(drop, 0.0, xc * scale0_ref[...][None])


def kernel(x, W1, W2):
    B, C, H, W = x.shape
    xt = jnp.transpose(x.reshape(B, C, H * W), (2, 0, 1))

    out_t = pl.pallas_call(
        _targetdrop_kernel,
        grid=(2 * _NCHUNK,),
        in_specs=[
            pl.BlockSpec((_CHUNK, B, C),
                         lambda j: (jnp.minimum(j, _NCHUNK - 1), 0, 0)),
            pl.BlockSpec((_D, C), lambda j: (0, 0)),
            pl.BlockSpec((C, _D), lambda j: (0, 0)),
        ],
        out_specs=pl.BlockSpec((_CHUNK, B, C),
                               lambda j: (jnp.maximum(j - _NCHUNK, 0),
                                          0, 0)),
        out_shape=jax.ShapeDtypeStruct((H * W, B, C), jnp.float32),
        scratch_shapes=[
            pltpu.VMEM((_HW, B, C), jnp.float32),
            pltpu.VMEM((B, C), jnp.float32),
            pltpu.VMEM((B, C), jnp.float32),
            pltpu.VMEM((B, C), jnp.int32),
            pltpu.VMEM((B, C), jnp.float32),
            pltpu.VMEM((B, C), jnp.int32),
            pltpu.VMEM((B, C), jnp.int32),
        ],
        compiler_params=pltpu.CompilerParams(
            dimension_semantics=("arbitrary",)),
    )(xt, W1, W2)

    return jnp.transpose(out_t, (1, 2, 0)).reshape(B, C, H, W)
